# initial kernel scaffold (unmeasured)
import jax
import jax.numpy as jnp
from jax import lax
from jax.experimental import pallas as pl
from jax.experimental.pallas import tpu as pltpu


def kernel(
    x,
):
    def body(*refs):
        pass

    out_shape = jax.ShapeDtypeStruct(..., jnp.float32)
    return pl.pallas_call(body, out_shape=out_shape)(...)


# baseline (device time: 104343 ns/iter reference)
import jax
import jax.numpy as jnp
from jax import lax
from jax.experimental import pallas as pl
from jax.experimental.pallas import tpu as pltpu


def kernel(x):
    m, n = x.shape

    def body(x_ref, out_ref, send_buf, recv_buf, send_sems, recv_sems):
        my_x = lax.axis_index("x")
        my_y = lax.axis_index("y")
        y_nbr = (my_x, 1 - my_y)
        x_nbr = (1 - my_x, my_y)

        barrier_sem = pltpu.get_barrier_semaphore()
        for nbr in (y_nbr, x_nbr):
            pl.semaphore_signal(
                barrier_sem, inc=1,
                device_id=nbr, device_id_type=pl.DeviceIdType.MESH,
            )
        pl.semaphore_wait(barrier_sem, 2)

        send_buf[...] = x_ref[...].astype(jnp.bfloat16)

        rdma1 = pltpu.make_async_remote_copy(
            src_ref=send_buf,
            dst_ref=recv_buf.at[0],
            send_sem=send_sems.at[0],
            recv_sem=recv_sems.at[0],
            device_id=y_nbr,
            device_id_type=pl.DeviceIdType.MESH,
        )
        rdma1.start()
        rdma1.wait()
        send_buf[...] = send_buf[...] + recv_buf[0]

        rdma2 = pltpu.make_async_remote_copy(
            src_ref=send_buf,
            dst_ref=recv_buf.at[1],
            send_sem=send_sems.at[1],
            recv_sem=recv_sems.at[1],
            device_id=x_nbr,
            device_id_type=pl.DeviceIdType.MESH,
        )
        rdma2.start()
        rdma2.wait()

        out_ref[...] = (
            send_buf[...].astype(jnp.float32)
            + recv_buf[1].astype(jnp.float32)
        )

    return pl.pallas_call(
        body,
        out_shape=jax.ShapeDtypeStruct((m, n), jnp.float32),
        in_specs=[pl.BlockSpec(memory_space=pltpu.VMEM)],
        out_specs=pl.BlockSpec(memory_space=pltpu.VMEM),
        scratch_shapes=[
            pltpu.VMEM((m, n), jnp.bfloat16),
            pltpu.VMEM((2, m, n), jnp.bfloat16),
            pltpu.SemaphoreType.DMA((2,)),
            pltpu.SemaphoreType.DMA((2,)),
        ],
        compiler_params=pltpu.CompilerParams(collective_id=0),
    )(x)


# device time: 59205 ns/iter; 1.7624x vs baseline; 1.7624x over previous
import jax
import jax.numpy as jnp
from jax import lax
from jax.experimental import pallas as pl
from jax.experimental.pallas import tpu as pltpu


def kernel(x):
    m, n = x.shape
    h = m // 2

    def body(x_ref, out_ref, buf_a, buf_b, recv, send_sems, recv_sems):
        my_x = lax.axis_index("x")
        my_y = lax.axis_index("y")
        y_nbr = (my_x, 1 - my_y)
        x_nbr = (1 - my_x, my_y)

        barrier_sem = pltpu.get_barrier_semaphore()
        for nbr in (y_nbr, x_nbr):
            pl.semaphore_signal(
                barrier_sem, inc=1,
                device_id=nbr, device_id_type=pl.DeviceIdType.MESH,
            )
        pl.semaphore_wait(barrier_sem, 2)

        buf_a[...] = x_ref[:h, :].astype(jnp.bfloat16)
        buf_b[...] = x_ref[h:, :].astype(jnp.bfloat16)

        def exchange(src, slot, nbr):
            return pltpu.make_async_remote_copy(
                src_ref=src,
                dst_ref=recv.at[slot],
                send_sem=send_sems.at[slot],
                recv_sem=recv_sems.at[slot],
                device_id=nbr,
                device_id_type=pl.DeviceIdType.MESH,
            )

        a1 = exchange(buf_a, 0, y_nbr)
        b1 = exchange(buf_b, 1, x_nbr)
        a1.start()
        b1.start()

        a1.wait()
        buf_a[...] = buf_a[...] + recv[0]
        a2 = exchange(buf_a, 2, x_nbr)
        a2.start()

        b1.wait()
        buf_b[...] = buf_b[...] + recv[1]
        b2 = exchange(buf_b, 3, y_nbr)
        b2.start()

        a2.wait()
        out_ref[:h, :] = buf_a[...].astype(jnp.float32) + recv[2].astype(
            jnp.float32
        )
        b2.wait()
        out_ref[h:, :] = buf_b[...].astype(jnp.float32) + recv[3].astype(
            jnp.float32
        )

    return pl.pallas_call(
        body,
        out_shape=jax.ShapeDtypeStruct((m, n), jnp.float32),
        in_specs=[pl.BlockSpec(memory_space=pltpu.VMEM)],
        out_specs=pl.BlockSpec(memory_space=pltpu.VMEM),
        scratch_shapes=[
            pltpu.VMEM((h, n), jnp.bfloat16),
            pltpu.VMEM((h, n), jnp.bfloat16),
            pltpu.VMEM((4, h, n), jnp.bfloat16),
            pltpu.SemaphoreType.DMA((4,)),
            pltpu.SemaphoreType.DMA((4,)),
        ],
        compiler_params=pltpu.CompilerParams(collective_id=0),
    )(x)


# device time: 49171 ns/iter; 2.1220x vs baseline; 1.2041x over previous
import jax
import jax.numpy as jnp
from jax import lax
from jax.experimental import pallas as pl
from jax.experimental.pallas import tpu as pltpu


def kernel(x):
    m, n = x.shape
    h = m // 2
    q = h // 2
    e = q // 2

    def body(x_ref, out_ref, abuf, bbuf, r1a, r1b, r2a, r2b,
             send_sems, recv_sems):
        dx = lax.axis_index("x")
        dy = lax.axis_index("y")
        y_nbr = (dx, 1 - dy)
        x_nbr = (1 - dx, dy)

        barrier_sem = pltpu.get_barrier_semaphore()
        for nbr in (y_nbr, x_nbr):
            pl.semaphore_signal(
                barrier_sem, inc=1,
                device_id=nbr, device_id_type=pl.DeviceIdType.MESH,
            )
        pl.semaphore_wait(barrier_sem, 2)

        def exch(src, dst, slot, nbr):
            return pltpu.make_async_remote_copy(
                src_ref=src, dst_ref=dst,
                send_sem=send_sems.at[slot], recv_sem=recv_sems.at[slot],
                device_id=nbr, device_id_type=pl.DeviceIdType.MESH,
            )

        a_keep = dy * q
        a_own = dy * q + dx * e
        b_keep = dx * q
        b_own = dx * q + dy * e

        abuf[...] = x_ref[:h, :].astype(jnp.bfloat16)
        a1 = exch(abuf.at[pl.ds((1 - dy) * q, q)], r1a, 0, y_nbr)
        a1.start()
        bbuf[...] = x_ref[h:, :].astype(jnp.bfloat16)
        b1 = exch(bbuf.at[pl.ds((1 - dx) * q, q)], r1b, 1, x_nbr)
        b1.start()

        a1.wait()
        abuf[pl.ds(a_keep, q), :] = abuf[pl.ds(a_keep, q), :] + r1a[...]
        a2 = exch(abuf.at[pl.ds(a_keep + (1 - dx) * e, e)], r2a, 2, x_nbr)
        a2.start()

        b1.wait()
        bbuf[pl.ds(b_keep, q), :] = bbuf[pl.ds(b_keep, q), :] + r1b[...]
        b2 = exch(bbuf.at[pl.ds(b_keep + (1 - dy) * e, e)], r2b, 3, y_nbr)
        b2.start()

        a2.wait()
        out_ref[pl.ds(a_own, e), :] = abuf[pl.ds(a_own, e), :] + r2a[...]
        a3 = exch(out_ref.at[pl.ds(a_own, e)],
                  out_ref.at[pl.ds(a_own, e)], 4, x_nbr)
        a3.start()

        b2.wait()
        out_ref[pl.ds(h + b_own, e), :] = (
            bbuf[pl.ds(b_own, e), :] + r2b[...]
        )
        b3 = exch(out_ref.at[pl.ds(h + b_own, e)],
                  out_ref.at[pl.ds(h + b_own, e)], 5, y_nbr)
        b3.start()

        a3.wait()
        a4 = exch(out_ref.at[pl.ds(dy * q, q)],
                  out_ref.at[pl.ds(dy * q, q)], 6, y_nbr)
        a4.start()

        b3.wait()
        b4 = exch(out_ref.at[pl.ds(h + dx * q, q)],
                  out_ref.at[pl.ds(h + dx * q, q)], 7, x_nbr)
        b4.start()

        a4.wait()
        b4.wait()

    return pl.pallas_call(
        body,
        out_shape=jax.ShapeDtypeStruct((m, n), jnp.bfloat16),
        in_specs=[pl.BlockSpec(memory_space=pltpu.VMEM)],
        out_specs=pl.BlockSpec(memory_space=pltpu.VMEM),
        scratch_shapes=[
            pltpu.VMEM((h, n), jnp.bfloat16),
            pltpu.VMEM((h, n), jnp.bfloat16),
            pltpu.VMEM((q, n), jnp.bfloat16),
            pltpu.VMEM((q, n), jnp.bfloat16),
            pltpu.VMEM((e, n), jnp.bfloat16),
            pltpu.VMEM((e, n), jnp.bfloat16),
            pltpu.SemaphoreType.DMA((8,)),
            pltpu.SemaphoreType.DMA((8,)),
        ],
        compiler_params=pltpu.CompilerParams(collective_id=0),
    )(x)


# device time: 45726 ns/iter; 2.2819x vs baseline; 1.0753x over previous
import jax
import jax.numpy as jnp
from jax import lax
from jax.experimental import pallas as pl
from jax.experimental.pallas import tpu as pltpu


def kernel(x):
    m, n = x.shape
    h = m // 2
    q = h // 2
    e = q // 2

    def body(x_ref, out_ref, abuf, bbuf,
             r1af, r1ak, r1bf, r1bk, r2a, r2b,
             send_sems, recv_sems):
        dx = lax.axis_index("x")
        dy = lax.axis_index("y")
        y_nbr = (dx, 1 - dy)
        x_nbr = (1 - dx, dy)

        barrier_sem = pltpu.get_barrier_semaphore()
        for nbr in (y_nbr, x_nbr):
            pl.semaphore_signal(
                barrier_sem, inc=1,
                device_id=nbr, device_id_type=pl.DeviceIdType.MESH,
            )
        pl.semaphore_wait(barrier_sem, 2)

        def exch(src, dst, slot, nbr):
            return pltpu.make_async_remote_copy(
                src_ref=src, dst_ref=dst,
                send_sem=send_sems.at[slot], recv_sem=recv_sems.at[slot],
                device_id=nbr, device_id_type=pl.DeviceIdType.MESH,
            )

        a_keep = dy * q
        a_fwd = a_keep + (1 - dx) * e
        a_own = a_keep + dx * e
        b_keep = dx * q
        b_fwd = b_keep + (1 - dy) * e
        b_own = b_keep + dy * e

        abuf[pl.ds((1 - dy) * q, q), :] = x_ref[
            pl.ds((1 - dy) * q, q), :
        ].astype(jnp.bfloat16)
        a1f = exch(abuf.at[pl.ds((1 - dy) * q + (1 - dx) * e, e)],
                   r1af, 0, y_nbr)
        a1f.start()
        bbuf[pl.ds((1 - dx) * q, q), :] = x_ref[
            pl.ds(h + (1 - dx) * q, q), :
        ].astype(jnp.bfloat16)
        b1f = exch(bbuf.at[pl.ds((1 - dx) * q + (1 - dy) * e, e)],
                   r1bf, 1, x_nbr)
        b1f.start()
        a1k = exch(abuf.at[pl.ds((1 - dy) * q + dx * e, e)], r1ak, 2, y_nbr)
        a1k.start()
        b1k = exch(bbuf.at[pl.ds((1 - dx) * q + dy * e, e)], r1bk, 3, x_nbr)
        b1k.start()
        abuf[pl.ds(a_keep, q), :] = x_ref[pl.ds(a_keep, q), :].astype(
            jnp.bfloat16
        )
        bbuf[pl.ds(b_keep, q), :] = x_ref[
            pl.ds(h + b_keep, q), :
        ].astype(jnp.bfloat16)

        a1f.wait()
        abuf[pl.ds(a_fwd, e), :] = abuf[pl.ds(a_fwd, e), :] + r1af[...]
        a2 = exch(abuf.at[pl.ds(a_fwd, e)], r2a, 4, x_nbr)
        a2.start()

        b1f.wait()
        bbuf[pl.ds(b_fwd, e), :] = bbuf[pl.ds(b_fwd, e), :] + r1bf[...]
        b2 = exch(bbuf.at[pl.ds(b_fwd, e)], r2b, 5, y_nbr)
        b2.start()

        a1k.wait()
        a2.wait()
        out_ref[pl.ds(a_own, e), :] = (
            abuf[pl.ds(a_own, e), :] + r1ak[...] + r2a[...]
        )
        a3 = exch(out_ref.at[pl.ds(a_own, e)],
                  out_ref.at[pl.ds(a_own, e)], 6, x_nbr)
        a3.start()

        b1k.wait()
        b2.wait()
        out_ref[pl.ds(h + b_own, e), :] = (
            bbuf[pl.ds(b_own, e), :] + r1bk[...] + r2b[...]
        )
        b3 = exch(out_ref.at[pl.ds(h + b_own, e)],
                  out_ref.at[pl.ds(h + b_own, e)], 7, y_nbr)
        b3.start()

        a4a = exch(out_ref.at[pl.ds(a_own, e)],
                   out_ref.at[pl.ds(a_own, e)], 8, y_nbr)
        a4a.start()
        b4a = exch(out_ref.at[pl.ds(h + b_own, e)],
                   out_ref.at[pl.ds(h + b_own, e)], 9, x_nbr)
        b4a.start()

        a3.wait()
        a4b = exch(out_ref.at[pl.ds(a_keep + (1 - dx) * e, e)],
                   out_ref.at[pl.ds(a_keep + (1 - dx) * e, e)], 10, y_nbr)
        a4b.start()
        b3.wait()
        b4b = exch(out_ref.at[pl.ds(h + b_keep + (1 - dy) * e, e)],
                   out_ref.at[pl.ds(h + b_keep + (1 - dy) * e, e)], 11, x_nbr)
        b4b.start()

        a4a.wait()
        b4a.wait()
        a4b.wait()
        b4b.wait()

    return pl.pallas_call(
        body,
        out_shape=jax.ShapeDtypeStruct((m, n), jnp.bfloat16),
        in_specs=[pl.BlockSpec(memory_space=pltpu.VMEM)],
        out_specs=pl.BlockSpec(memory_space=pltpu.VMEM),
        scratch_shapes=[
            pltpu.VMEM((h, n), jnp.bfloat16),
            pltpu.VMEM((h, n), jnp.bfloat16),
            pltpu.VMEM((e, n), jnp.bfloat16),
            pltpu.VMEM((e, n), jnp.bfloat16),
            pltpu.VMEM((e, n), jnp.bfloat16),
            pltpu.VMEM((e, n), jnp.bfloat16),
            pltpu.VMEM((e, n), jnp.bfloat16),
            pltpu.VMEM((e, n), jnp.bfloat16),
            pltpu.SemaphoreType.DMA((12,)),
            pltpu.SemaphoreType.DMA((12,)),
        ],
        compiler_params=pltpu.CompilerParams(collective_id=0),
    )(x)


# device time: 8537 ns/iter; 12.2224x vs baseline; 5.3562x over previous
import contextlib
import os

import jax
import jax.numpy as jnp
from jax import lax
from jax.experimental import pallas as pl
from jax.experimental.pallas import tpu as pltpu

_SCOPES = os.environ.get("KERNEL_SCOPES") == "1"


def _scope(name):
    return jax.named_scope(name) if _SCOPES else contextlib.nullcontext()


def kernel(x):
    m, n = x.shape
    h = m // 2
    q = h // 2
    e = q // 2

    def body(x_ref, out_ref, abuf, bbuf,
             r1af, r1ak, r1bf, r1bk, r2a, r2b,
             send_sems, recv_sems):
        dx = lax.axis_index("x")
        dy = lax.axis_index("y")
        y_nbr = (dx, 1 - dy)
        x_nbr = (1 - dx, dy)

        with _scope("barrier"):
            barrier_sem = pltpu.get_barrier_semaphore()
            for nbr in (y_nbr, x_nbr):
                pl.semaphore_signal(
                    barrier_sem, inc=1,
                    device_id=nbr, device_id_type=pl.DeviceIdType.MESH,
                )
            pl.semaphore_wait(barrier_sem, 2)

        def exch(src, dst, slot, nbr):
            return pltpu.make_async_remote_copy(
                src_ref=src, dst_ref=dst,
                send_sem=send_sems.at[slot], recv_sem=recv_sems.at[slot],
                device_id=nbr, device_id_type=pl.DeviceIdType.MESH,
            )

        a_keep = dy * q
        a_fwd = a_keep + (1 - dx) * e
        a_own = a_keep + dx * e
        b_keep = dx * q
        b_fwd = b_keep + (1 - dy) * e
        b_own = b_keep + dy * e

        with _scope("cast_send"):
            abuf[pl.ds((1 - dy) * q, q), :] = x_ref[
                pl.ds((1 - dy) * q, q), :
            ].astype(jnp.bfloat16)
            a1f = exch(abuf.at[pl.ds((1 - dy) * q + (1 - dx) * e, e)],
                       r1af, 0, y_nbr)
            a1f.start()
            bbuf[pl.ds((1 - dx) * q, q), :] = x_ref[
                pl.ds(h + (1 - dx) * q, q), :
            ].astype(jnp.bfloat16)
            b1f = exch(bbuf.at[pl.ds((1 - dx) * q + (1 - dy) * e, e)],
                       r1bf, 1, x_nbr)
            b1f.start()
            a1k = exch(abuf.at[pl.ds((1 - dy) * q + dx * e, e)],
                       r1ak, 2, y_nbr)
            a1k.start()
            b1k = exch(bbuf.at[pl.ds((1 - dx) * q + dy * e, e)],
                       r1bk, 3, x_nbr)
            b1k.start()
        with _scope("wait_a1f"):
            a1f.wait()
        with _scope("add_afwd"):
            abuf[pl.ds(a_fwd, e), :] = (
                x_ref[pl.ds(a_fwd, e), :] + r1af[...].astype(jnp.float32)
            ).astype(jnp.bfloat16)
            a2 = exch(abuf.at[pl.ds(a_fwd, e)], r2a, 4, x_nbr)
            a2.start()

        with _scope("wait_b1f"):
            b1f.wait()
        with _scope("add_bfwd"):
            bbuf[pl.ds(b_fwd, e), :] = (
                x_ref[pl.ds(h + b_fwd, e), :] + r1bf[...].astype(jnp.float32)
            ).astype(jnp.bfloat16)
            b2 = exch(bbuf.at[pl.ds(b_fwd, e)], r2b, 5, y_nbr)
            b2.start()

        with _scope("wait_a1k_a2"):
            a1k.wait()
            a2.wait()
        with _scope("store_aown"):
            out_ref[pl.ds(a_own, e), :] = (
                x_ref[pl.ds(a_own, e), :]
                + (r1ak[...] + r2a[...]).astype(jnp.float32)
            ).astype(jnp.bfloat16)
            a3 = exch(out_ref.at[pl.ds(a_own, e)],
                      out_ref.at[pl.ds(a_own, e)], 6, x_nbr)
            a3.start()

        with _scope("wait_b1k_b2"):
            b1k.wait()
            b2.wait()
        with _scope("store_bown"):
            out_ref[pl.ds(h + b_own, e), :] = (
                x_ref[pl.ds(h + b_own, e), :]
                + (r1bk[...] + r2b[...]).astype(jnp.float32)
            ).astype(jnp.bfloat16)
            b3 = exch(out_ref.at[pl.ds(h + b_own, e)],
                      out_ref.at[pl.ds(h + b_own, e)], 7, y_nbr)
            b3.start()

        with _scope("start_ag_own"):
            a4a = exch(out_ref.at[pl.ds(a_own, e)],
                       out_ref.at[pl.ds(a_own, e)], 8, y_nbr)
            a4a.start()
            b4a = exch(out_ref.at[pl.ds(h + b_own, e)],
                       out_ref.at[pl.ds(h + b_own, e)], 9, x_nbr)
            b4a.start()

        with _scope("wait_a3"):
            a3.wait()
        with _scope("start_a4b"):
            a4b = exch(out_ref.at[pl.ds(a_keep + (1 - dx) * e, e)],
                       out_ref.at[pl.ds(a_keep + (1 - dx) * e, e)], 10, y_nbr)
            a4b.start()
        with _scope("wait_b3"):
            b3.wait()
        with _scope("start_b4b"):
            b4b = exch(out_ref.at[pl.ds(h + b_keep + (1 - dy) * e, e)],
                       out_ref.at[pl.ds(h + b_keep + (1 - dy) * e, e)],
                       11, x_nbr)
            b4b.start()

        with _scope("wait_tail"):
            a4a.wait()
            b4a.wait()
            a4b.wait()
            b4b.wait()

    return pl.pallas_call(
        body,
        out_shape=jax.ShapeDtypeStruct((m, n), jnp.bfloat16),
        in_specs=[pl.BlockSpec(memory_space=pltpu.VMEM)],
        out_specs=pl.BlockSpec(memory_space=pltpu.VMEM),
        scratch_shapes=[
            pltpu.VMEM((h, n), jnp.bfloat16),
            pltpu.VMEM((h, n), jnp.bfloat16),
            pltpu.VMEM((e, n), jnp.bfloat16),
            pltpu.VMEM((e, n), jnp.bfloat16),
            pltpu.VMEM((e, n), jnp.bfloat16),
            pltpu.VMEM((e, n), jnp.bfloat16),
            pltpu.VMEM((e, n), jnp.bfloat16),
            pltpu.VMEM((e, n), jnp.bfloat16),
            pltpu.SemaphoreType.DMA((12,)),
            pltpu.SemaphoreType.DMA((12,)),
        ],
        compiler_params=pltpu.CompilerParams(collective_id=0),
    )(x)
